# baseline (speedup 1.0000x reference)
import dataclasses
import functools

import jax
import jax.numpy as jnp
from jax import lax
from jax.experimental import pallas as pl
from jax.experimental.pallas import tpu as pltpu
from jax.experimental.pallas import tpu_sc as plsc

F = 32
NN = 50000
NE = 800000
NW = 32
LANES = 16

RPW1 = 1568
NOUT = 5000
RPW2 = 160

EPW = NE // NW
WIN = 1000
NVREG = 63
CAP_CH = EPW // 128 + 1
SLOTS = CAP_CH * 128
GWIN = 640
SENT = 0xFFFF


def _mesh():
    return plsc.VectorSubcoreMesh(core_axis_name="c", subcore_axis_name="s")


def _sc_params(**kw):
    cp = pltpu.CompilerParams()
    if "needs_layout_passes" in pltpu.CompilerParams.__dataclass_fields__:
        cp = dataclasses.replace(cp, needs_layout_passes=False)
    if kw:
        cp = dataclasses.replace(cp, **kw)
    return cp


def _vgather(x, idx):
    return lax.gather(
        x, idx[:, None],
        dimension_numbers=lax.GatherDimensionNumbers(
            offset_dims=(), collapsed_slice_dims=(0,), start_index_map=(0,)),
        slice_sizes=(1,),
        mode=lax.GatherScatterMode.PROMISE_IN_BOUNDS)


def _bin_edges(dvals, mode):
    out_types = (
        jax.ShapeDtypeStruct((NW * NW * SLOTS,), jnp.int32),
        jax.ShapeDtypeStruct((NW * NW * SLOTS,), jnp.int32),
        jax.ShapeDtypeStruct((NW, NW), jnp.int32),
    )

    @functools.partial(
        pl.kernel, out_type=out_types, mesh=_mesh(),
        compiler_params=_sc_params(),
        scratch_types=[
            pltpu.VMEM((1008,), jnp.int32),
            pltpu.VMEM((NW, 144), jnp.int32),
            pltpu.VMEM((NW, 144), jnp.int32),
            pltpu.VMEM((NW,), jnp.int32),
            pltpu.VMEM((NW,), jnp.int32),
        ])
    def kern(d_hbm, bd_hbm, be_hbm, cnt_hbm, didx, stg_d, stg_e, cntv, flshv):
        wid = lax.axis_index("s") * 2 + lax.axis_index("c")
        ebase = wid * EPW
        lanes = lax.iota(jnp.int32, LANES)
        zeros = jnp.zeros((LANES,), jnp.int32)

        for i in range(2):
            cntv[pl.ds(i * 16, 16)] = zeros
            flshv[pl.ds(i * 16, 16)] = zeros

        @pl.loop(0, EPW // WIN)
        def _win(win):
            wbase = ebase + win * WIN
            pltpu.sync_copy(d_hbm.at[pl.ds(wbase, WIN)], didx.at[pl.ds(0, WIN)])

            @pl.loop(0, NVREG)
            def _v(v):
                d = didx[pl.ds(v * 16, 16)]
                nvalid = jnp.where(v == NVREG - 1, 8, 16)
                vmask = lanes < nvalid
                if mode == 1:
                    key = d
                    q = (d * 2674) >> 22
                    owner = q + jnp.where(d >= (q + 1) * RPW1, 1, 0)
                    keep = vmask
                else:
                    du = d.astype(jnp.uint32)
                    q10 = ((du * jnp.uint32(52429)) >> jnp.uint32(19)).astype(jnp.int32)
                    keep = vmask & ((d - q10 * 10) == 9)
                    key = q10
                    owner = ((key >> 5) * 6554) >> 15
                packed = jnp.where(
                    keep,
                    (owner << 20) | (key << 4) | lanes,
                    (31 << 20) | (SENT << 4) | lanes)
                srt, _ = plsc.sort_key_val(packed, packed)
                owner_s = srt >> 20
                key_s = (srt >> 4) & SENT
                lane_s = srt & 15
                valid_s = key_s != SENT
                nxt = _vgather(srt, (lanes + 1) & 15)
                is_last = ((lanes == 15) | ((nxt >> 20) != owner_s)
                           | (((nxt >> 4) & SENT) == SENT))
                prev_owner = _vgather(srt, (lanes - 1) & 15) >> 20
                bnd = (lanes == 0) | (prev_owner != owner_s)
                run_start = plsc.cummax(jnp.where(bnd, lanes, 0))
                occ = lanes - run_start
                base = plsc.load_gather(cntv, [owner_s])
                pos = base + occ
                eid_s = wbase + v * 16 + lane_s
                plsc.store_scatter(stg_d, [owner_s, pos], key_s, mask=valid_s)
                plsc.store_scatter(stg_e, [owner_s, pos], eid_s, mask=valid_s)
                upd = is_last & valid_s
                plsc.addupdate_scatter(cntv, [owner_s], occ + 1, mask=upd)
                maxcnt = jnp.max(jnp.where(upd, pos + 1, 0))

                @pl.when(maxcnt >= 128)
                def _flush():
                    for o in range(NW):
                        hsl = pl.ds((o // 16) * 16, 16)
                        omask = lanes == (o % 16)
                        hv = cntv[hsl]
                        cnt_o = jnp.max(jnp.where(omask, hv, 0))

                        @pl.when(cnt_o >= 128)
                        def _fo(o=o, hsl=hsl, omask=omask, hv=hv):
                            fv = flshv[hsl]
                            nfl = jnp.max(jnp.where(omask, fv, 0))
                            off = (o * NW + wid) * SLOTS + nfl * 128
                            pltpu.sync_copy(stg_d.at[o, pl.ds(0, 128)],
                                            bd_hbm.at[pl.ds(off, 128)])
                            pltpu.sync_copy(stg_e.at[o, pl.ds(0, 128)],
                                            be_hbm.at[pl.ds(off, 128)])
                            stg_d[o, pl.ds(0, 16)] = stg_d[o, pl.ds(128, 16)]
                            stg_e[o, pl.ds(0, 16)] = stg_e[o, pl.ds(128, 16)]
                            cntv[hsl] = hv - jnp.where(omask, 128, 0)
                            flshv[hsl] = fv + jnp.where(omask, 1, 0)

        sentv = jnp.full((LANES,), SENT, jnp.int32)
        for o in range(NW):
            hsl = pl.ds((o // 16) * 16, 16)
            omask = lanes == (o % 16)
            hv = cntv[hsl]
            cnt_o = jnp.max(jnp.where(omask, hv, 0))

            @pl.when(cnt_o > 0)
            def _drain(o=o, hsl=hsl, omask=omask, cnt_o=cnt_o):
                for t in range(8):
                    pad = (t * 16 + lanes) >= cnt_o
                    stg_d[o, pl.ds(t * 16, 16)] = jnp.where(
                        pad, sentv, stg_d[o, pl.ds(t * 16, 16)])
                    stg_e[o, pl.ds(t * 16, 16)] = jnp.where(
                        pad, zeros, stg_e[o, pl.ds(t * 16, 16)])
                fv = flshv[hsl]
                nfl = jnp.max(jnp.where(omask, fv, 0))
                off = (o * NW + wid) * SLOTS + nfl * 128
                pltpu.sync_copy(stg_d.at[o, pl.ds(0, 128)],
                                bd_hbm.at[pl.ds(off, 128)])
                pltpu.sync_copy(stg_e.at[o, pl.ds(0, 128)],
                                be_hbm.at[pl.ds(off, 128)])
                flshv[hsl] = fv + jnp.where(omask, 1, 0)

        pltpu.sync_copy(flshv, cnt_hbm.at[wid])

    return kern(dvals)


def _gather_pairs(a_tab4, b_tab4, dst, src):
    nwin = NE // GWIN

    tmax = (nwin + NW - 1) // NW

    @functools.partial(
        pl.kernel,
        out_type=(jax.ShapeDtypeStruct((NE, F), jnp.float32),
                  jax.ShapeDtypeStruct((NE, F), jnp.float32)),
        mesh=_mesh(),
        compiler_params=_sc_params(use_tc_tiling_on_sc=False),
        scratch_types=[
            [pltpu.VMEM((GWIN,), jnp.int32) for _ in range(2)],
            [pltpu.VMEM((GWIN,), jnp.int32) for _ in range(2)],
            [pltpu.VMEM((GWIN, F), jnp.float32) for _ in range(2)],
            [pltpu.VMEM((GWIN, F), jnp.float32) for _ in range(2)],
            pltpu.SemaphoreType.DMA,
            pltpu.SemaphoreType.DMA,
            pltpu.SemaphoreType.DMA,
        ])
    def kern(a_hbm, b_hbm, d_hbm, s_hbm, ga_hbm, gb_hbm, di, si, ar, br,
             semi, semg, semw):
        wid = lax.axis_index("s") * 2 + lax.axis_index("c")
        nt = (nwin - wid + NW - 1) // NW

        def base_of(t):
            return (wid + t * NW) * GWIN

        def fire_idx(t, p):
            base = base_of(t)
            pltpu.async_copy(d_hbm.at[pl.ds(base, GWIN)], di[p], semi)
            pltpu.async_copy(s_hbm.at[pl.ds(base, GWIN)], si[p], semi)

        def drain_idx(p):
            pltpu.make_async_copy(d_hbm.at[pl.ds(0, GWIN)], di[p], semi).wait()
            pltpu.make_async_copy(s_hbm.at[pl.ds(0, GWIN)], si[p], semi).wait()

        def fire_gathers(p):
            for j in range(GWIN // 128):
                sl = pl.ds(j * 128, 128)
                pltpu.async_copy(a_hbm.at[di[p].at[sl]], ar[p].at[sl], semg)
                pltpu.async_copy(b_hbm.at[si[p].at[sl]], br[p].at[sl], semg)

        def drain_gathers(p):
            for j in range(GWIN // 128):
                sl = pl.ds(j * 128, 128)
                pltpu.make_async_copy(ga_hbm.at[pl.ds(0, 128)], ar[p].at[sl],
                                      semg).wait()
                pltpu.make_async_copy(ga_hbm.at[pl.ds(0, 128)], br[p].at[sl],
                                      semg).wait()

        def fire_writes(t, p):
            base = base_of(t)
            pltpu.async_copy(ar[p], ga_hbm.at[pl.ds(base, GWIN)], semw)
            pltpu.async_copy(br[p], gb_hbm.at[pl.ds(base, GWIN)], semw)

        def drain_writes(p):
            pltpu.make_async_copy(ar[p], ga_hbm.at[pl.ds(0, GWIN)], semw).wait()
            pltpu.make_async_copy(br[p], gb_hbm.at[pl.ds(0, GWIN)], semw).wait()

        @pl.when(nt > 0)
        def _pro():
            fire_idx(0, 0)

        @pl.loop(0, (tmax + 1) // 2)
        def _t2(t2):
            for half in range(2):
                p = half
                t = t2 * 2 + half

                @pl.when(t < nt)
                def _(t=t, p=p):
                    @pl.when(t >= 2)
                    def _():
                        drain_writes(p)
                    drain_idx(p)
                    fire_gathers(p)

                    @pl.when(t + 1 < nt)
                    def _():
                        fire_idx(t + 1, 1 - p)
                    drain_gathers(p)
                    fire_writes(t, p)

        @pl.when(nt >= 1)
        def _ep1():
            drain_writes(0)

        @pl.when(nt >= 2)
        def _ep2():
            drain_writes(1)

    return kern(jnp.reshape(a_tab4, (NN, F)), jnp.reshape(b_tab4, (NN, F)),
                dst, src)


def _seg_max(m4, bkeys, beids, counts, mode):
    rpw = RPW1 if mode == 1 else RPW2
    ntot = NN if mode == 1 else NOUT
    nlast = ntot - (NW - 1) * rpw

    @functools.partial(
        pl.kernel,
        out_type=jax.ShapeDtypeStruct((ntot, F), jnp.float32),
        mesh=_mesh(),
        compiler_params=_sc_params(use_tc_tiling_on_sc=False),
        scratch_types=[
            pltpu.VMEM((rpw, F), jnp.float32),
            pltpu.VMEM((128,), jnp.int32),
            pltpu.VMEM((128,), jnp.int32),
            pltpu.VMEM((128, F), jnp.float32),
            pltpu.VMEM((NW * NW,), jnp.int32),
            pltpu.SemaphoreType.DMA,
        ])
    def kern(m_hbm, bd_hbm, be_hbm, c_hbm, o_hbm, acc, dbuf, ebuf, mrows,
             cbuf, sem):
        wid = lax.axis_index("s") * 2 + lax.axis_index("c")
        lanes = lax.iota(jnp.int32, LANES)
        lo = wid * rpw
        hi = jnp.minimum(lo + rpw, ntot)
        span = hi - lo
        ninf = jnp.full((LANES,), -jnp.inf, jnp.float32)

        @pl.loop(0, rpw)
        def _init(r):
            acc[r, pl.ds(0, 16)] = ninf
            acc[r, pl.ds(16, 16)] = ninf

        pltpu.sync_copy(c_hbm, cbuf)

        @pl.loop(0, NW)
        def _w(w):
            cv = plsc.load_gather(cbuf, [zeros16 := jnp.zeros((LANES,), jnp.int32) + (w * NW + wid)])
            nc = jnp.max(cv)

            @pl.loop(0, nc)
            def _c(c):
                off = (wid * NW + w) * SLOTS + c * 128
                pltpu.sync_copy(bd_hbm.at[pl.ds(off, 128)], dbuf)
                pltpu.sync_copy(be_hbm.at[pl.ds(off, 128)], ebuf)
                pltpu.async_copy(m_hbm.at[ebuf], mrows, sem).wait()
                for g in range(8):
                    d = dbuf[pl.ds(g * 16, 16)]
                    loc = d - lo
                    okm = (loc >= 0) & (loc < span)
                    packed = jnp.where(okm, (loc << 4) | lanes,
                                       0x7FFF0 | lanes)
                    srt, _ = plsc.sort_key_val(packed, packed)
                    loc_s = srt >> 4
                    lane_s = srt & 15
                    ok_s = loc_s < span
                    prev = _vgather(srt, (lanes - 1) & 15) >> 4
                    bnd = (lanes == 0) | (prev != loc_s)
                    run_start = plsc.cummax(jnp.where(bnd, lanes, 0))
                    occ = lanes - run_start
                    row = g * 16 + lane_s
                    locc = jnp.where(ok_s, loc_s, 0)
                    npass = jnp.max(jnp.where(ok_s, occ, 0)) + 1

                    @pl.loop(0, npass)
                    def _p(k, locc=locc, occ=occ, ok_s=ok_s, row=row):
                        sel = ok_s & (occ == k)
                        for f in range(F):
                            fv = jnp.zeros((LANES,), jnp.int32) + f
                            cur = plsc.load_gather(acc, [locc, fv], mask=sel)
                            upd = plsc.load_gather(mrows, [row, fv], mask=sel)
                            plsc.store_scatter(acc, [locc, fv],
                                               jnp.maximum(cur, upd), mask=sel)

        @pl.when(wid < NW - 1)
        def _wb():
            pltpu.sync_copy(acc, o_hbm.at[pl.ds(lo, rpw)])

        @pl.when(wid == NW - 1)
        def _wbl():
            pltpu.sync_copy(acc.at[pl.ds(0, nlast)], o_hbm.at[pl.ds(lo, nlast)])

    return jnp.reshape(kern(jnp.reshape(m4, (NE, F)), bkeys, beids, counts),
                       (ntot // 4, 4 * F))


def _nodeproj128(h4raw, wd4, ws4, b4, fix):
    blk4 = 512

    def body(h_ref, wd_ref, ws_ref, b_ref, a_ref, bo_ref):
        x = h_ref[...]
        if fix:
            x = jnp.where(x == -jnp.inf, 0.0, x)
        a_ref[...] = jnp.dot(x, wd_ref[...],
                             preferred_element_type=jnp.float32) + b_ref[...]
        bo_ref[...] = jnp.dot(x, ws_ref[...],
                              preferred_element_type=jnp.float32)

    n4 = NN // 4
    return pl.pallas_call(
        body,
        grid=((n4 + blk4 - 1) // blk4,),
        in_specs=[pl.BlockSpec((blk4, 4 * F), lambda i: (i, 0)),
                  pl.BlockSpec((4 * F, 4 * F), lambda i: (0, 0)),
                  pl.BlockSpec((4 * F, 4 * F), lambda i: (0, 0)),
                  pl.BlockSpec((1, 4 * F), lambda i: (0, 0))],
        out_specs=[pl.BlockSpec((blk4, 4 * F), lambda i: (i, 0)),
                   pl.BlockSpec((blk4, 4 * F), lambda i: (i, 0))],
        out_shape=(jax.ShapeDtypeStruct((n4, 4 * F), jnp.float32),
                   jax.ShapeDtypeStruct((n4, 4 * F), jnp.float32)),
    )(h4raw, wd4, ws4, b4)


def _edge_mlp(ga4, gb4, ef4, we4, w24, b24):
    blk4 = 1000
    ne4 = NE // 4

    def body(ga_ref, gb_ref, ef_ref, we_ref, w2_ref, b2_ref, m_ref):
        pre = ga_ref[...] + gb_ref[...] + jnp.dot(
            ef_ref[...], we_ref[...], preferred_element_type=jnp.float32)
        m_ref[...] = jnp.dot(jnp.maximum(pre, 0.0), w2_ref[...],
                             preferred_element_type=jnp.float32) + b2_ref[...]

    return pl.pallas_call(
        body,
        grid=(ne4 // blk4,),
        in_specs=[pl.BlockSpec((blk4, 4 * F), lambda i: (i, 0)),
                  pl.BlockSpec((blk4, 4 * F), lambda i: (i, 0)),
                  pl.BlockSpec((blk4, 24), lambda i: (i, 0)),
                  pl.BlockSpec((24, 4 * F), lambda i: (0, 0)),
                  pl.BlockSpec((4 * F, 4 * F), lambda i: (0, 0)),
                  pl.BlockSpec((1, 4 * F), lambda i: (0, 0))],
        out_specs=pl.BlockSpec((blk4, 4 * F), lambda i: (i, 0)),
        out_shape=jax.ShapeDtypeStruct((ne4, 4 * F), jnp.float32),
    )(ga4, gb4, ef4, we4, w24, b24)


def _regression(h2c4, wr4, br4):
    n4 = NOUT // 4

    def body(x_ref, w_ref, b_ref, o_ref):
        x = x_ref[...]
        x = jnp.where(x == -jnp.inf, 0.0, x)
        o_ref[...] = jnp.dot(x, w_ref[...],
                             preferred_element_type=jnp.float32) + b_ref[...]

    y4 = pl.pallas_call(
        body,
        grid=(1,),
        in_specs=[pl.BlockSpec((n4, 4 * F), lambda i: (0, 0)),
                  pl.BlockSpec((4 * F, 4 * F), lambda i: (0, 0)),
                  pl.BlockSpec((1, 4 * F), lambda i: (0, 0))],
        out_specs=pl.BlockSpec((n4, 4 * F), lambda i: (0, 0)),
        out_shape=jax.ShapeDtypeStruct((n4, 4 * F), jnp.float32),
    )(h2c4, wr4, br4)
    return jnp.reshape(y4, (NOUT, F))


def kernel(h, edge_index, edge_features, W1a, b1a, W2a, b2a,
           W1b, b1b, W2b, b2b, Wr, br):
    ei = edge_index.astype(jnp.int32)
    src1, dst1, src2, dst2 = ei[0], ei[1], ei[2], ei[3]
    eye4 = jnp.eye(4, dtype=jnp.float32)
    ef24 = edge_features.reshape(NE // 4, 24)
    wz3 = jnp.zeros((3, F), jnp.float32)
    wd1, ws1 = W1a[:F], W1a[F:2 * F]
    we41 = jnp.kron(eye4, jnp.concatenate([W1a[2 * F:], wz3], axis=0))
    w241 = jnp.kron(eye4, W2a)
    b241 = jnp.tile(b2a, 4).reshape(1, 4 * F)
    wd42 = jnp.kron(eye4, W1b[:F])
    ws42 = jnp.kron(eye4, W1b[F:2 * F])
    b142 = jnp.tile(b1b, 4).reshape(1, 4 * F)
    we42 = jnp.kron(eye4, jnp.concatenate([wz3, W1b[2 * F:]], axis=0))
    w242 = jnp.kron(eye4, W2b)
    b242 = jnp.tile(b2b, 4).reshape(1, 4 * F)
    wr4 = jnp.kron(eye4, Wr)
    br4 = jnp.tile(br, 4).reshape(1, 4 * F)

    bk1, be1, cn1 = _bin_edges(dst1, 1)
    bk2, be2, cn2 = _bin_edges(dst2, 2)

    ne4 = NE // 4

    wd41 = jnp.kron(eye4, wd1)
    ws41 = jnp.kron(eye4, ws1)
    b141 = jnp.tile(b1a, 4).reshape(1, 4 * F)
    h4 = jnp.reshape(h, (NN // 4, 4 * F))
    a14, b14 = _nodeproj128(h4, wd41, ws41, b141, False)
    ga1, gb1 = _gather_pairs(a14, b14, dst1, src1)
    m14 = _edge_mlp(jnp.reshape(ga1, (ne4, 4 * F)),
                    jnp.reshape(gb1, (ne4, 4 * F)), ef24, we41, w241, b241)
    h14 = _seg_max(m14, bk1, be1, cn1.reshape(-1), 1)

    a24, b24 = _nodeproj128(h14, wd42, ws42, b142, True)
    ga2, gb2 = _gather_pairs(a24, b24, dst2, src2)
    m24 = _edge_mlp(jnp.reshape(ga2, (ne4, 4 * F)),
                    jnp.reshape(gb2, (ne4, 4 * F)), ef24, we42, w242, b242)
    h2c4 = _seg_max(m24, bk2, be2, cn2.reshape(-1), 2)

    return _regression(h2c4, wr4, br4)


# speedup vs baseline: 1.1046x; 1.1046x over previous
import dataclasses
import functools

import jax
import jax.numpy as jnp
from jax import lax
from jax.experimental import pallas as pl
from jax.experimental.pallas import tpu as pltpu
from jax.experimental.pallas import tpu_sc as plsc

F = 32
NN = 50000
NE = 800000
NW = 32
LANES = 16

RPW1 = 1568
NOUT = 5000
RPW2 = 160

EPW = NE // NW
WIN = 1000
NVREG = 63
CAP_CH = EPW // 128 + 1
SLOTS = CAP_CH * 128
GWIN = 640
SENT = 0xFFFF


def _mesh():
    return plsc.VectorSubcoreMesh(core_axis_name="c", subcore_axis_name="s")


def _sc_params(**kw):
    cp = pltpu.CompilerParams()
    if "needs_layout_passes" in pltpu.CompilerParams.__dataclass_fields__:
        cp = dataclasses.replace(cp, needs_layout_passes=False)
    if kw:
        cp = dataclasses.replace(cp, **kw)
    return cp


def _vgather(x, idx):
    return lax.gather(
        x, idx[:, None],
        dimension_numbers=lax.GatherDimensionNumbers(
            offset_dims=(), collapsed_slice_dims=(0,), start_index_map=(0,)),
        slice_sizes=(1,),
        mode=lax.GatherScatterMode.PROMISE_IN_BOUNDS)


def _bin_edges(dvals, mode):
    out_types = (
        jax.ShapeDtypeStruct((NW * NW * SLOTS,), jnp.int32),
        jax.ShapeDtypeStruct((NW * NW * SLOTS,), jnp.int32),
        jax.ShapeDtypeStruct((NW, NW), jnp.int32),
    )

    @functools.partial(
        pl.kernel, out_type=out_types, mesh=_mesh(),
        compiler_params=_sc_params(),
        scratch_types=[
            pltpu.VMEM((1008,), jnp.int32),
            pltpu.VMEM((NW, 144), jnp.int32),
            pltpu.VMEM((NW, 144), jnp.int32),
            pltpu.VMEM((NW,), jnp.int32),
            pltpu.VMEM((NW,), jnp.int32),
        ])
    def kern(d_hbm, bd_hbm, be_hbm, cnt_hbm, didx, stg_d, stg_e, cntv, flshv):
        wid = lax.axis_index("s") * 2 + lax.axis_index("c")
        ebase = wid * EPW
        lanes = lax.iota(jnp.int32, LANES)
        zeros = jnp.zeros((LANES,), jnp.int32)

        for i in range(2):
            cntv[pl.ds(i * 16, 16)] = zeros
            flshv[pl.ds(i * 16, 16)] = zeros

        @pl.loop(0, EPW // WIN)
        def _win(win):
            wbase = ebase + win * WIN
            pltpu.sync_copy(d_hbm.at[pl.ds(wbase, WIN)], didx.at[pl.ds(0, WIN)])

            @pl.loop(0, NVREG)
            def _v(v):
                d = didx[pl.ds(v * 16, 16)]
                nvalid = jnp.where(v == NVREG - 1, 8, 16)
                vmask = lanes < nvalid
                if mode == 1:
                    key = d
                    q = (d * 2674) >> 22
                    owner = q + jnp.where(d >= (q + 1) * RPW1, 1, 0)
                    keep = vmask
                else:
                    du = d.astype(jnp.uint32)
                    q10 = ((du * jnp.uint32(52429)) >> jnp.uint32(19)).astype(jnp.int32)
                    keep = vmask & ((d - q10 * 10) == 9)
                    key = q10
                    owner = ((key >> 5) * 6554) >> 15
                packed = jnp.where(
                    keep,
                    (owner << 20) | (key << 4) | lanes,
                    (31 << 20) | (SENT << 4) | lanes)
                srt, _ = plsc.sort_key_val(packed, packed)
                owner_s = srt >> 20
                key_s = (srt >> 4) & SENT
                lane_s = srt & 15
                valid_s = key_s != SENT
                nxt = _vgather(srt, (lanes + 1) & 15)
                is_last = ((lanes == 15) | ((nxt >> 20) != owner_s)
                           | (((nxt >> 4) & SENT) == SENT))
                prev_owner = _vgather(srt, (lanes - 1) & 15) >> 20
                bnd = (lanes == 0) | (prev_owner != owner_s)
                run_start = plsc.cummax(jnp.where(bnd, lanes, 0))
                occ = lanes - run_start
                base = plsc.load_gather(cntv, [owner_s])
                pos = base + occ
                eid_s = wbase + v * 16 + lane_s
                plsc.store_scatter(stg_d, [owner_s, pos], key_s, mask=valid_s)
                plsc.store_scatter(stg_e, [owner_s, pos], eid_s, mask=valid_s)
                upd = is_last & valid_s
                plsc.addupdate_scatter(cntv, [owner_s], occ + 1, mask=upd)
                maxcnt = jnp.max(jnp.where(upd, pos + 1, 0))

                @pl.when(maxcnt >= 128)
                def _flush():
                    for o in range(NW):
                        hsl = pl.ds((o // 16) * 16, 16)
                        omask = lanes == (o % 16)
                        hv = cntv[hsl]
                        cnt_o = jnp.max(jnp.where(omask, hv, 0))

                        @pl.when(cnt_o >= 128)
                        def _fo(o=o, hsl=hsl, omask=omask, hv=hv):
                            fv = flshv[hsl]
                            nfl = jnp.max(jnp.where(omask, fv, 0))
                            off = (o * NW + wid) * SLOTS + nfl * 128
                            pltpu.sync_copy(stg_d.at[o, pl.ds(0, 128)],
                                            bd_hbm.at[pl.ds(off, 128)])
                            pltpu.sync_copy(stg_e.at[o, pl.ds(0, 128)],
                                            be_hbm.at[pl.ds(off, 128)])
                            stg_d[o, pl.ds(0, 16)] = stg_d[o, pl.ds(128, 16)]
                            stg_e[o, pl.ds(0, 16)] = stg_e[o, pl.ds(128, 16)]
                            cntv[hsl] = hv - jnp.where(omask, 128, 0)
                            flshv[hsl] = fv + jnp.where(omask, 1, 0)

        sentv = jnp.full((LANES,), SENT, jnp.int32)
        for o in range(NW):
            hsl = pl.ds((o // 16) * 16, 16)
            omask = lanes == (o % 16)
            hv = cntv[hsl]
            cnt_o = jnp.max(jnp.where(omask, hv, 0))

            @pl.when(cnt_o > 0)
            def _drain(o=o, hsl=hsl, omask=omask, cnt_o=cnt_o):
                for t in range(8):
                    pad = (t * 16 + lanes) >= cnt_o
                    stg_d[o, pl.ds(t * 16, 16)] = jnp.where(
                        pad, sentv, stg_d[o, pl.ds(t * 16, 16)])
                    stg_e[o, pl.ds(t * 16, 16)] = jnp.where(
                        pad, zeros, stg_e[o, pl.ds(t * 16, 16)])
                fv = flshv[hsl]
                nfl = jnp.max(jnp.where(omask, fv, 0))
                off = (o * NW + wid) * SLOTS + nfl * 128
                pltpu.sync_copy(stg_d.at[o, pl.ds(0, 128)],
                                bd_hbm.at[pl.ds(off, 128)])
                pltpu.sync_copy(stg_e.at[o, pl.ds(0, 128)],
                                be_hbm.at[pl.ds(off, 128)])
                flshv[hsl] = fv + jnp.where(omask, 1, 0)

        pltpu.sync_copy(flshv, cnt_hbm.at[wid])

    return kern(dvals)


def _gather_pairs(a_tab4, b_tab4, dst, src):
    nwin = NE // GWIN

    tmax = (nwin + NW - 1) // NW

    @functools.partial(
        pl.kernel,
        out_type=(jax.ShapeDtypeStruct((NE, F), jnp.float32),
                  jax.ShapeDtypeStruct((NE, F), jnp.float32)),
        mesh=_mesh(),
        compiler_params=_sc_params(use_tc_tiling_on_sc=False),
        scratch_types=[
            [pltpu.VMEM((GWIN,), jnp.int32) for _ in range(2)],
            [pltpu.VMEM((GWIN,), jnp.int32) for _ in range(2)],
            [pltpu.VMEM((GWIN, F), jnp.float32) for _ in range(2)],
            [pltpu.VMEM((GWIN, F), jnp.float32) for _ in range(2)],
            pltpu.SemaphoreType.DMA,
            pltpu.SemaphoreType.DMA,
            pltpu.SemaphoreType.DMA,
        ])
    def kern(a_hbm, b_hbm, d_hbm, s_hbm, ga_hbm, gb_hbm, di, si, ar, br,
             semi, semg, semw):
        wid = lax.axis_index("s") * 2 + lax.axis_index("c")
        nt = (nwin - wid + NW - 1) // NW

        def base_of(t):
            return (wid + t * NW) * GWIN

        def fire_idx(t, p):
            base = base_of(t)
            pltpu.async_copy(d_hbm.at[pl.ds(base, GWIN)], di[p], semi)
            pltpu.async_copy(s_hbm.at[pl.ds(base, GWIN)], si[p], semi)

        def drain_idx(p):
            pltpu.make_async_copy(d_hbm.at[pl.ds(0, GWIN)], di[p], semi).wait()
            pltpu.make_async_copy(s_hbm.at[pl.ds(0, GWIN)], si[p], semi).wait()

        def fire_gathers(p):
            for j in range(GWIN // 128):
                sl = pl.ds(j * 128, 128)
                pltpu.async_copy(a_hbm.at[di[p].at[sl]], ar[p].at[sl], semg)
                pltpu.async_copy(b_hbm.at[si[p].at[sl]], br[p].at[sl], semg)

        def drain_gathers(p):
            for j in range(GWIN // 128):
                sl = pl.ds(j * 128, 128)
                pltpu.make_async_copy(ga_hbm.at[pl.ds(0, 128)], ar[p].at[sl],
                                      semg).wait()
                pltpu.make_async_copy(ga_hbm.at[pl.ds(0, 128)], br[p].at[sl],
                                      semg).wait()

        def fire_writes(t, p):
            base = base_of(t)
            pltpu.async_copy(ar[p], ga_hbm.at[pl.ds(base, GWIN)], semw)
            pltpu.async_copy(br[p], gb_hbm.at[pl.ds(base, GWIN)], semw)

        def drain_writes(p):
            pltpu.make_async_copy(ar[p], ga_hbm.at[pl.ds(0, GWIN)], semw).wait()
            pltpu.make_async_copy(br[p], gb_hbm.at[pl.ds(0, GWIN)], semw).wait()

        @pl.when(nt > 0)
        def _pro():
            fire_idx(0, 0)

        @pl.loop(0, (tmax + 1) // 2)
        def _t2(t2):
            for half in range(2):
                p = half
                t = t2 * 2 + half

                @pl.when(t < nt)
                def _(t=t, p=p):
                    @pl.when(t >= 2)
                    def _():
                        drain_writes(p)
                    drain_idx(p)
                    fire_gathers(p)

                    @pl.when(t + 1 < nt)
                    def _():
                        fire_idx(t + 1, 1 - p)
                    drain_gathers(p)
                    fire_writes(t, p)

        @pl.when(nt >= 1)
        def _ep1():
            drain_writes(0)

        @pl.when(nt >= 2)
        def _ep2():
            drain_writes(1)

    return kern(jnp.reshape(a_tab4, (NN, F)), jnp.reshape(b_tab4, (NN, F)),
                dst, src)


def _seg_max(m4, bkeys, beids, counts, mode):
    rpw = RPW1 if mode == 1 else RPW2
    ntot = NN if mode == 1 else NOUT
    nlast = ntot - (NW - 1) * rpw

    @functools.partial(
        pl.kernel,
        out_type=jax.ShapeDtypeStruct((ntot, F), jnp.float32),
        mesh=_mesh(),
        compiler_params=_sc_params(use_tc_tiling_on_sc=False),
        scratch_types=[
            pltpu.VMEM((rpw, F), jnp.float32),
            pltpu.VMEM((128,), jnp.int32),
            pltpu.VMEM((128,), jnp.int32),
            pltpu.VMEM((128, F), jnp.float32),
            pltpu.VMEM((NW * NW,), jnp.int32),
            pltpu.SemaphoreType.DMA,
        ])
    def kern(m_hbm, bd_hbm, be_hbm, c_hbm, o_hbm, acc, dbuf, ebuf, mrows,
             cbuf, sem):
        wid = lax.axis_index("s") * 2 + lax.axis_index("c")
        lanes = lax.iota(jnp.int32, LANES)
        lo = wid * rpw
        hi = jnp.minimum(lo + rpw, ntot)
        span = hi - lo
        ninf = jnp.full((LANES,), -jnp.inf, jnp.float32)

        @pl.loop(0, rpw)
        def _init(r):
            acc[r, pl.ds(0, 16)] = ninf
            acc[r, pl.ds(16, 16)] = ninf

        pltpu.sync_copy(c_hbm, cbuf)

        @pl.loop(0, NW)
        def _w(w):
            cv = plsc.load_gather(cbuf, [zeros16 := jnp.zeros((LANES,), jnp.int32) + (w * NW + wid)])
            nc = jnp.max(cv)

            @pl.loop(0, nc)
            def _c(c):
                off = (wid * NW + w) * SLOTS + c * 128
                pltpu.sync_copy(bd_hbm.at[pl.ds(off, 128)], dbuf)
                pltpu.sync_copy(be_hbm.at[pl.ds(off, 128)], ebuf)
                pltpu.async_copy(m_hbm.at[ebuf], mrows, sem).wait()
                for g in range(8):
                    d = dbuf[pl.ds(g * 16, 16)]
                    loc = d - lo
                    okm = (loc >= 0) & (loc < span)
                    packed = jnp.where(okm, (loc << 4) | lanes,
                                       0x7FFF0 | lanes)
                    srt, _ = plsc.sort_key_val(packed, packed)
                    loc_s = srt >> 4
                    lane_s = srt & 15
                    ok_s = loc_s < span
                    prev = _vgather(srt, (lanes - 1) & 15) >> 4
                    bnd = (lanes == 0) | (prev != loc_s)
                    run_start = plsc.cummax(jnp.where(bnd, lanes, 0))
                    occ = lanes - run_start
                    row = g * 16 + lane_s
                    locc = jnp.where(ok_s, loc_s, 0)
                    npass = jnp.max(jnp.where(ok_s, occ, 0)) + 1

                    @pl.loop(0, npass)
                    def _p(k, locc=locc, occ=occ, ok_s=ok_s, row=row):
                        sel = ok_s & (occ == k)
                        for f in range(F):
                            fv = jnp.zeros((LANES,), jnp.int32) + f
                            cur = plsc.load_gather(acc, [locc, fv], mask=sel)
                            upd = plsc.load_gather(mrows, [row, fv], mask=sel)
                            plsc.store_scatter(acc, [locc, fv],
                                               jnp.maximum(cur, upd), mask=sel)

        @pl.when(wid < NW - 1)
        def _wb():
            pltpu.sync_copy(acc, o_hbm.at[pl.ds(lo, rpw)])

        @pl.when(wid == NW - 1)
        def _wbl():
            pltpu.sync_copy(acc.at[pl.ds(0, nlast)], o_hbm.at[pl.ds(lo, nlast)])

    return jnp.reshape(kern(jnp.reshape(m4, (NE, F)), bkeys, beids, counts),
                       (ntot // 4, 4 * F))


def _nodeproj128(h4raw, wd4, ws4, b4, fix):
    blk4 = 512

    def body(h_ref, wd_ref, ws_ref, b_ref, a_ref, bo_ref):
        x = h_ref[...]
        if fix:
            x = jnp.where(x == -jnp.inf, 0.0, x)
        a_ref[...] = jnp.dot(x, wd_ref[...],
                             preferred_element_type=jnp.float32) + b_ref[...]
        bo_ref[...] = jnp.dot(x, ws_ref[...],
                              preferred_element_type=jnp.float32)

    n4 = NN // 4
    return pl.pallas_call(
        body,
        grid=((n4 + blk4 - 1) // blk4,),
        in_specs=[pl.BlockSpec((blk4, 4 * F), lambda i: (i, 0)),
                  pl.BlockSpec((4 * F, 4 * F), lambda i: (0, 0)),
                  pl.BlockSpec((4 * F, 4 * F), lambda i: (0, 0)),
                  pl.BlockSpec((1, 4 * F), lambda i: (0, 0))],
        out_specs=[pl.BlockSpec((blk4, 4 * F), lambda i: (i, 0)),
                   pl.BlockSpec((blk4, 4 * F), lambda i: (i, 0))],
        out_shape=(jax.ShapeDtypeStruct((n4, 4 * F), jnp.float32),
                   jax.ShapeDtypeStruct((n4, 4 * F), jnp.float32)),
    )(h4raw, wd4, ws4, b4)


def _edge_mlp(ga4, gb4, ef4, we4, w24, b24):
    blk4 = 1000
    ne4 = NE // 4

    def body(ga_ref, gb_ref, ef_ref, we_ref, w2_ref, b2_ref, m_ref):
        pre = ga_ref[...] + gb_ref[...] + jnp.dot(
            ef_ref[...], we_ref[...], preferred_element_type=jnp.float32)
        m_ref[...] = jnp.dot(jnp.maximum(pre, 0.0), w2_ref[...],
                             preferred_element_type=jnp.float32) + b2_ref[...]

    return pl.pallas_call(
        body,
        grid=(ne4 // blk4,),
        in_specs=[pl.BlockSpec((blk4, 4 * F), lambda i: (i, 0)),
                  pl.BlockSpec((blk4, 4 * F), lambda i: (i, 0)),
                  pl.BlockSpec((blk4, F), lambda i: (i, 0)),
                  pl.BlockSpec((F, 4 * F), lambda i: (0, 0)),
                  pl.BlockSpec((4 * F, 4 * F), lambda i: (0, 0)),
                  pl.BlockSpec((1, 4 * F), lambda i: (0, 0))],
        out_specs=pl.BlockSpec((blk4, 4 * F), lambda i: (i, 0)),
        out_shape=jax.ShapeDtypeStruct((ne4, 4 * F), jnp.float32),
    )(ga4, gb4, ef4, we4, w24, b24)


def _regression(h2c4, wr4, br4):
    n4 = NOUT // 4

    def body(x_ref, w_ref, b_ref, o_ref):
        x = x_ref[...]
        x = jnp.where(x == -jnp.inf, 0.0, x)
        o_ref[...] = jnp.dot(x, w_ref[...],
                             preferred_element_type=jnp.float32) + b_ref[...]

    y4 = pl.pallas_call(
        body,
        grid=(1,),
        in_specs=[pl.BlockSpec((n4, 4 * F), lambda i: (0, 0)),
                  pl.BlockSpec((4 * F, 4 * F), lambda i: (0, 0)),
                  pl.BlockSpec((1, 4 * F), lambda i: (0, 0))],
        out_specs=pl.BlockSpec((n4, 4 * F), lambda i: (0, 0)),
        out_shape=jax.ShapeDtypeStruct((n4, 4 * F), jnp.float32),
    )(h2c4, wr4, br4)
    return jnp.reshape(y4, (NOUT, F))


def kernel(h, edge_index, edge_features, W1a, b1a, W2a, b2a,
           W1b, b1b, W2b, b2b, Wr, br):
    ei = edge_index.astype(jnp.int32)
    src1, dst1, src2, dst2 = ei[0], ei[1], ei[2], ei[3]
    eye4 = jnp.eye(4, dtype=jnp.float32)
    efr = edge_features.reshape(NE // 4, 8, 3)
    ef04 = jnp.pad(efr[:, 0::2, :], ((0, 0), (0, 0), (0, 5))).reshape(NE // 4, F)
    wz = jnp.zeros((5, F), jnp.float32)
    wd1, ws1 = W1a[:F], W1a[F:2 * F]
    we41 = jnp.kron(eye4, jnp.concatenate([W1a[2 * F:], wz], axis=0))
    w241 = jnp.kron(eye4, W2a)
    b241 = jnp.tile(b2a, 4).reshape(1, 4 * F)
    wd42 = jnp.kron(eye4, W1b[:F])
    ws42 = jnp.kron(eye4, W1b[F:2 * F])
    b142 = jnp.tile(b1b, 4).reshape(1, 4 * F)
    we42 = jnp.kron(eye4, jnp.concatenate([W1b[2 * F:], wz], axis=0))
    w242 = jnp.kron(eye4, W2b)
    b242 = jnp.tile(b2b, 4).reshape(1, 4 * F)
    wr4 = jnp.kron(eye4, Wr)
    br4 = jnp.tile(br, 4).reshape(1, 4 * F)

    bk1, be1, cn1 = _bin_edges(dst1, 1)
    bk2, be2, cn2 = _bin_edges(dst2, 2)

    ne4 = NE // 4

    wd41 = jnp.kron(eye4, wd1)
    ws41 = jnp.kron(eye4, ws1)
    b141 = jnp.tile(b1a, 4).reshape(1, 4 * F)
    h4 = jnp.reshape(h, (NN // 4, 4 * F))
    a14, b14 = _nodeproj128(h4, wd41, ws41, b141, False)
    ga1, gb1 = _gather_pairs(a14, b14, dst1, src1)
    m14 = _edge_mlp(jnp.reshape(ga1, (ne4, 4 * F)),
                    jnp.reshape(gb1, (ne4, 4 * F)), ef04, we41, w241, b241)
    h14 = _seg_max(m14, bk1, be1, cn1.reshape(-1), 1)

    efr2, _ = lax.optimization_barrier((efr, h14))
    ef14 = jnp.pad(efr2[:, 1::2, :], ((0, 0), (0, 0), (0, 5))).reshape(NE // 4, F)

    a24, b24 = _nodeproj128(h14, wd42, ws42, b142, True)
    ga2, gb2 = _gather_pairs(a24, b24, dst2, src2)
    m24 = _edge_mlp(jnp.reshape(ga2, (ne4, 4 * F)),
                    jnp.reshape(gb2, (ne4, 4 * F)), ef14, we42, w242, b242)
    h2c4 = _seg_max(m24, bk2, be2, cn2.reshape(-1), 2)

    return _regression(h2c4, wr4, br4)


# speedup vs baseline: 1.3718x; 1.2418x over previous
import dataclasses
import functools

import jax
import jax.numpy as jnp
from jax import lax
from jax.experimental import pallas as pl
from jax.experimental.pallas import tpu as pltpu
from jax.experimental.pallas import tpu_sc as plsc

F = 32
NN = 50000
NE = 800000
NW = 32
LANES = 16

RPW1 = 1568
NOUT = 5000
RPW2 = 160

EPW = NE // NW
WIN = 1000
NVREG = 63
CAP_CH = EPW // 128 + 1
SLOTS = CAP_CH * 128
GWIN = 640
SENT = 0xFFFF


def _mesh():
    return plsc.VectorSubcoreMesh(core_axis_name="c", subcore_axis_name="s")


def _sc_params(**kw):
    cp = pltpu.CompilerParams()
    if "needs_layout_passes" in pltpu.CompilerParams.__dataclass_fields__:
        cp = dataclasses.replace(cp, needs_layout_passes=False)
    if kw:
        cp = dataclasses.replace(cp, **kw)
    return cp


def _vgather(x, idx):
    return lax.gather(
        x, idx[:, None],
        dimension_numbers=lax.GatherDimensionNumbers(
            offset_dims=(), collapsed_slice_dims=(0,), start_index_map=(0,)),
        slice_sizes=(1,),
        mode=lax.GatherScatterMode.PROMISE_IN_BOUNDS)


def _bin_edges(dvals, mode):
    out_types = (
        jax.ShapeDtypeStruct((NW * NW * SLOTS,), jnp.int32),
        jax.ShapeDtypeStruct((NW * NW * SLOTS,), jnp.int32),
        jax.ShapeDtypeStruct((NW, NW), jnp.int32),
    )

    @functools.partial(
        pl.kernel, out_type=out_types, mesh=_mesh(),
        compiler_params=_sc_params(),
        scratch_types=[
            pltpu.VMEM((1008,), jnp.int32),
            pltpu.VMEM((NW, 144), jnp.int32),
            pltpu.VMEM((NW, 144), jnp.int32),
            pltpu.VMEM((NW,), jnp.int32),
            pltpu.VMEM((NW,), jnp.int32),
        ])
    def kern(d_hbm, bd_hbm, be_hbm, cnt_hbm, didx, stg_d, stg_e, cntv, flshv):
        wid = lax.axis_index("s") * 2 + lax.axis_index("c")
        ebase = wid * EPW
        lanes = lax.iota(jnp.int32, LANES)
        zeros = jnp.zeros((LANES,), jnp.int32)

        for i in range(2):
            cntv[pl.ds(i * 16, 16)] = zeros
            flshv[pl.ds(i * 16, 16)] = zeros

        @pl.loop(0, EPW // WIN)
        def _win(win):
            wbase = ebase + win * WIN
            pltpu.sync_copy(d_hbm.at[pl.ds(wbase, WIN)], didx.at[pl.ds(0, WIN)])

            @pl.loop(0, NVREG)
            def _v(v):
                d = didx[pl.ds(v * 16, 16)]
                nvalid = jnp.where(v == NVREG - 1, 8, 16)
                vmask = lanes < nvalid
                if mode == 1:
                    key = d
                    q = (d * 2674) >> 22
                    owner = q + jnp.where(d >= (q + 1) * RPW1, 1, 0)
                    keep = vmask
                else:
                    du = d.astype(jnp.uint32)
                    q10 = ((du * jnp.uint32(52429)) >> jnp.uint32(19)).astype(jnp.int32)
                    keep = vmask & ((d - q10 * 10) == 9)
                    key = q10
                    owner = ((key >> 5) * 6554) >> 15
                packed = jnp.where(
                    keep,
                    (owner << 20) | (key << 4) | lanes,
                    (31 << 20) | (SENT << 4) | lanes)
                srt, _ = plsc.sort_key_val(packed, packed)
                owner_s = srt >> 20
                key_s = (srt >> 4) & SENT
                lane_s = srt & 15
                valid_s = key_s != SENT
                nxt = _vgather(srt, (lanes + 1) & 15)
                is_last = ((lanes == 15) | ((nxt >> 20) != owner_s)
                           | (((nxt >> 4) & SENT) == SENT))
                prev_owner = _vgather(srt, (lanes - 1) & 15) >> 20
                bnd = (lanes == 0) | (prev_owner != owner_s)
                run_start = plsc.cummax(jnp.where(bnd, lanes, 0))
                occ = lanes - run_start
                base = plsc.load_gather(cntv, [owner_s])
                pos = base + occ
                eid_s = wbase + v * 16 + lane_s
                plsc.store_scatter(stg_d, [owner_s, pos], key_s, mask=valid_s)
                plsc.store_scatter(stg_e, [owner_s, pos], eid_s, mask=valid_s)
                upd = is_last & valid_s
                plsc.addupdate_scatter(cntv, [owner_s], occ + 1, mask=upd)
                maxcnt = jnp.max(jnp.where(upd, pos + 1, 0))

                @pl.when(maxcnt >= 128)
                def _flush():
                    for o in range(NW):
                        hsl = pl.ds((o // 16) * 16, 16)
                        omask = lanes == (o % 16)
                        hv = cntv[hsl]
                        cnt_o = jnp.max(jnp.where(omask, hv, 0))

                        @pl.when(cnt_o >= 128)
                        def _fo(o=o, hsl=hsl, omask=omask, hv=hv):
                            fv = flshv[hsl]
                            nfl = jnp.max(jnp.where(omask, fv, 0))
                            off = (o * NW + wid) * SLOTS + nfl * 128
                            pltpu.sync_copy(stg_d.at[o, pl.ds(0, 128)],
                                            bd_hbm.at[pl.ds(off, 128)])
                            pltpu.sync_copy(stg_e.at[o, pl.ds(0, 128)],
                                            be_hbm.at[pl.ds(off, 128)])
                            stg_d[o, pl.ds(0, 16)] = stg_d[o, pl.ds(128, 16)]
                            stg_e[o, pl.ds(0, 16)] = stg_e[o, pl.ds(128, 16)]
                            cntv[hsl] = hv - jnp.where(omask, 128, 0)
                            flshv[hsl] = fv + jnp.where(omask, 1, 0)

        sentv = jnp.full((LANES,), SENT, jnp.int32)
        for o in range(NW):
            hsl = pl.ds((o // 16) * 16, 16)
            omask = lanes == (o % 16)
            hv = cntv[hsl]
            cnt_o = jnp.max(jnp.where(omask, hv, 0))

            @pl.when(cnt_o > 0)
            def _drain(o=o, hsl=hsl, omask=omask, cnt_o=cnt_o):
                for t in range(8):
                    pad = (t * 16 + lanes) >= cnt_o
                    stg_d[o, pl.ds(t * 16, 16)] = jnp.where(
                        pad, sentv, stg_d[o, pl.ds(t * 16, 16)])
                    stg_e[o, pl.ds(t * 16, 16)] = jnp.where(
                        pad, zeros, stg_e[o, pl.ds(t * 16, 16)])
                fv = flshv[hsl]
                nfl = jnp.max(jnp.where(omask, fv, 0))
                off = (o * NW + wid) * SLOTS + nfl * 128
                pltpu.sync_copy(stg_d.at[o, pl.ds(0, 128)],
                                bd_hbm.at[pl.ds(off, 128)])
                pltpu.sync_copy(stg_e.at[o, pl.ds(0, 128)],
                                be_hbm.at[pl.ds(off, 128)])
                flshv[hsl] = fv + jnp.where(omask, 1, 0)

        pltpu.sync_copy(flshv, cnt_hbm.at[wid])

    return kern(dvals)


def _gather_pairs(a_tab4, b_tab4, dst, src):
    nwin = NE // GWIN

    tmax = (nwin + NW - 1) // NW

    @functools.partial(
        pl.kernel,
        out_type=(jax.ShapeDtypeStruct((NE, F), jnp.float32),
                  jax.ShapeDtypeStruct((NE, F), jnp.float32)),
        mesh=_mesh(),
        compiler_params=_sc_params(use_tc_tiling_on_sc=False),
        scratch_types=[
            [pltpu.VMEM((GWIN,), jnp.int32) for _ in range(2)],
            [pltpu.VMEM((GWIN,), jnp.int32) for _ in range(2)],
            [pltpu.VMEM((GWIN, F), jnp.float32) for _ in range(2)],
            [pltpu.VMEM((GWIN, F), jnp.float32) for _ in range(2)],
            pltpu.SemaphoreType.DMA,
            pltpu.SemaphoreType.DMA,
            pltpu.SemaphoreType.DMA,
        ])
    def kern(a_hbm, b_hbm, d_hbm, s_hbm, ga_hbm, gb_hbm, di, si, ar, br,
             semi, semg, semw):
        wid = lax.axis_index("s") * 2 + lax.axis_index("c")
        nt = (nwin - wid + NW - 1) // NW

        def base_of(t):
            return (wid + t * NW) * GWIN

        def fire_idx(t, p):
            base = base_of(t)
            pltpu.async_copy(d_hbm.at[pl.ds(base, GWIN)], di[p], semi)
            pltpu.async_copy(s_hbm.at[pl.ds(base, GWIN)], si[p], semi)

        def drain_idx(p):
            pltpu.make_async_copy(d_hbm.at[pl.ds(0, GWIN)], di[p], semi).wait()
            pltpu.make_async_copy(s_hbm.at[pl.ds(0, GWIN)], si[p], semi).wait()

        def fire_gathers(p):
            for j in range(GWIN // 128):
                sl = pl.ds(j * 128, 128)
                pltpu.async_copy(a_hbm.at[di[p].at[sl]], ar[p].at[sl], semg)
                pltpu.async_copy(b_hbm.at[si[p].at[sl]], br[p].at[sl], semg)

        def drain_gathers(p):
            for j in range(GWIN // 128):
                sl = pl.ds(j * 128, 128)
                pltpu.make_async_copy(ga_hbm.at[pl.ds(0, 128)], ar[p].at[sl],
                                      semg).wait()
                pltpu.make_async_copy(ga_hbm.at[pl.ds(0, 128)], br[p].at[sl],
                                      semg).wait()

        def fire_writes(t, p):
            base = base_of(t)
            pltpu.async_copy(ar[p], ga_hbm.at[pl.ds(base, GWIN)], semw)
            pltpu.async_copy(br[p], gb_hbm.at[pl.ds(base, GWIN)], semw)

        def drain_writes(p):
            pltpu.make_async_copy(ar[p], ga_hbm.at[pl.ds(0, GWIN)], semw).wait()
            pltpu.make_async_copy(br[p], gb_hbm.at[pl.ds(0, GWIN)], semw).wait()

        @pl.when(nt > 0)
        def _pro():
            fire_idx(0, 0)

        @pl.loop(0, (tmax + 1) // 2)
        def _t2(t2):
            for half in range(2):
                p = half
                t = t2 * 2 + half

                @pl.when(t < nt)
                def _(t=t, p=p):
                    @pl.when(t >= 2)
                    def _():
                        drain_writes(p)
                    drain_idx(p)
                    fire_gathers(p)

                    @pl.when(t + 1 < nt)
                    def _():
                        fire_idx(t + 1, 1 - p)
                    drain_gathers(p)
                    fire_writes(t, p)

        @pl.when(nt >= 1)
        def _ep1():
            drain_writes(0)

        @pl.when(nt >= 2)
        def _ep2():
            drain_writes(1)

    return kern(jnp.reshape(a_tab4, (NN, F)), jnp.reshape(b_tab4, (NN, F)),
                dst, src)


def _seg_max(m4, bkeys, beids, counts, mode):
    rpw = RPW1 if mode == 1 else RPW2
    ntot = NN if mode == 1 else NOUT
    nlast = ntot - (NW - 1) * rpw

    @functools.partial(
        pl.kernel,
        out_type=jax.ShapeDtypeStruct((ntot, F), jnp.float32),
        mesh=_mesh(),
        compiler_params=_sc_params(use_tc_tiling_on_sc=False),
        scratch_types=[
            pltpu.VMEM((rpw, F), jnp.float32),
            pltpu.VMEM((128,), jnp.int32),
            pltpu.VMEM((128,), jnp.int32),
            pltpu.VMEM((128, F), jnp.float32),
            pltpu.VMEM((NW * NW,), jnp.int32),
            pltpu.SemaphoreType.DMA,
        ])
    def kern(m_hbm, bd_hbm, be_hbm, c_hbm, o_hbm, acc, dbuf, ebuf, mrows,
             cbuf, sem):
        wid = lax.axis_index("s") * 2 + lax.axis_index("c")
        lanes = lax.iota(jnp.int32, LANES)
        lo = wid * rpw
        hi = jnp.minimum(lo + rpw, ntot)
        span = hi - lo
        ninf = jnp.full((LANES,), -jnp.inf, jnp.float32)

        @pl.loop(0, rpw)
        def _init(r):
            acc[r, pl.ds(0, 16)] = ninf
            acc[r, pl.ds(16, 16)] = ninf

        pltpu.sync_copy(c_hbm, cbuf)

        @pl.loop(0, NW)
        def _w(w):
            cv = plsc.load_gather(cbuf, [zeros16 := jnp.zeros((LANES,), jnp.int32) + (w * NW + wid)])
            nc = jnp.max(cv)

            @pl.loop(0, nc)
            def _c(c):
                off = (wid * NW + w) * SLOTS + c * 128
                pltpu.sync_copy(bd_hbm.at[pl.ds(off, 128)], dbuf)
                pltpu.sync_copy(be_hbm.at[pl.ds(off, 128)], ebuf)
                pltpu.async_copy(m_hbm.at[ebuf], mrows, sem).wait()
                for g in range(8):
                    d = dbuf[pl.ds(g * 16, 16)]
                    loc = d - lo
                    okm = (loc >= 0) & (loc < span)
                    packed = jnp.where(okm, (loc << 4) | lanes,
                                       0x7FFF0 | lanes)
                    srt, _ = plsc.sort_key_val(packed, packed)
                    loc_s = srt >> 4
                    lane_s = srt & 15
                    ok_s = loc_s < span
                    prev = _vgather(srt, (lanes - 1) & 15) >> 4
                    bnd = (lanes == 0) | (prev != loc_s)
                    run_start = plsc.cummax(jnp.where(bnd, lanes, 0))
                    occ = lanes - run_start
                    row = g * 16 + lane_s
                    locc = jnp.where(ok_s, loc_s, 0)
                    npass = jnp.max(jnp.where(ok_s, occ, 0)) + 1

                    @pl.loop(0, npass)
                    def _p(k, locc=locc, occ=occ, ok_s=ok_s, row=row):
                        sel = ok_s & (occ == k)
                        for f in range(F):
                            fv = jnp.zeros((LANES,), jnp.int32) + f
                            cur = plsc.load_gather(acc, [locc, fv], mask=sel)
                            upd = plsc.load_gather(mrows, [row, fv], mask=sel)
                            plsc.store_scatter(acc, [locc, fv],
                                               jnp.maximum(cur, upd), mask=sel)

        @pl.when(wid < NW - 1)
        def _wb():
            pltpu.sync_copy(acc, o_hbm.at[pl.ds(lo, rpw)])

        @pl.when(wid == NW - 1)
        def _wbl():
            pltpu.sync_copy(acc.at[pl.ds(0, nlast)], o_hbm.at[pl.ds(lo, nlast)])

    return jnp.reshape(kern(jnp.reshape(m4, (NE, F)), bkeys, beids, counts),
                       (ntot // 4, 4 * F))


def _nodeproj128(h4raw, wd4, ws4, b4, fix):
    blk4 = 512

    def body(h_ref, wd_ref, ws_ref, b_ref, a_ref, bo_ref):
        x = h_ref[...]
        if fix:
            x = jnp.where(x == -jnp.inf, 0.0, x)
        a_ref[...] = jnp.dot(x, wd_ref[...],
                             preferred_element_type=jnp.float32) + b_ref[...]
        bo_ref[...] = jnp.dot(x, ws_ref[...],
                              preferred_element_type=jnp.float32)

    n4 = NN // 4
    return pl.pallas_call(
        body,
        grid=((n4 + blk4 - 1) // blk4,),
        in_specs=[pl.BlockSpec((blk4, 4 * F), lambda i: (i, 0)),
                  pl.BlockSpec((4 * F, 4 * F), lambda i: (0, 0)),
                  pl.BlockSpec((4 * F, 4 * F), lambda i: (0, 0)),
                  pl.BlockSpec((1, 4 * F), lambda i: (0, 0))],
        out_specs=[pl.BlockSpec((blk4, 4 * F), lambda i: (i, 0)),
                   pl.BlockSpec((blk4, 4 * F), lambda i: (i, 0))],
        out_shape=(jax.ShapeDtypeStruct((n4, 4 * F), jnp.float32),
                   jax.ShapeDtypeStruct((n4, 4 * F), jnp.float32)),
    )(h4raw, wd4, ws4, b4)


def _edge_mlp(ga4, gb4, ef4, we4, w24, b24):
    blk4 = 1000
    ne4 = NE // 4

    def body(ga_ref, gb_ref, ef_ref, we_ref, w2_ref, b2_ref, m_ref):
        pre = ga_ref[...] + gb_ref[...] + jnp.dot(
            ef_ref[...], we_ref[...], preferred_element_type=jnp.float32)
        m_ref[...] = jnp.dot(jnp.maximum(pre, 0.0), w2_ref[...],
                             preferred_element_type=jnp.float32) + b2_ref[...]

    return pl.pallas_call(
        body,
        grid=(ne4 // blk4,),
        in_specs=[pl.BlockSpec((blk4, 4 * F), lambda i: (i, 0)),
                  pl.BlockSpec((blk4, 4 * F), lambda i: (i, 0)),
                  pl.BlockSpec((blk4, F), lambda i: (i, 0)),
                  pl.BlockSpec((F, 4 * F), lambda i: (0, 0)),
                  pl.BlockSpec((4 * F, 4 * F), lambda i: (0, 0)),
                  pl.BlockSpec((1, 4 * F), lambda i: (0, 0))],
        out_specs=pl.BlockSpec((blk4, 4 * F), lambda i: (i, 0)),
        out_shape=jax.ShapeDtypeStruct((ne4, 4 * F), jnp.float32),
    )(ga4, gb4, ef4, we4, w24, b24)


def _regression(h2c4, wr4, br4):
    n4 = NOUT // 4

    def body(x_ref, w_ref, b_ref, o_ref):
        x = x_ref[...]
        x = jnp.where(x == -jnp.inf, 0.0, x)
        o_ref[...] = jnp.dot(x, w_ref[...],
                             preferred_element_type=jnp.float32) + b_ref[...]

    y4 = pl.pallas_call(
        body,
        grid=(1,),
        in_specs=[pl.BlockSpec((n4, 4 * F), lambda i: (0, 0)),
                  pl.BlockSpec((4 * F, 4 * F), lambda i: (0, 0)),
                  pl.BlockSpec((1, 4 * F), lambda i: (0, 0))],
        out_specs=pl.BlockSpec((n4, 4 * F), lambda i: (0, 0)),
        out_shape=jax.ShapeDtypeStruct((n4, 4 * F), jnp.float32),
    )(h2c4, wr4, br4)
    return jnp.reshape(y4, (NOUT, F))


def kernel(h, edge_index, edge_features, W1a, b1a, W2a, b2a,
           W1b, b1b, W2b, b2b, Wr, br):
    ei = edge_index.astype(jnp.int32)
    src1, dst1, src2, dst2 = ei[0], ei[1], ei[2], ei[3]
    eye4 = jnp.eye(4, dtype=jnp.float32)
    efr = edge_features.reshape(NE // 4, 8, 3)
    ef04 = jnp.pad(efr[:, 0::2, :], ((0, 0), (0, 0), (0, 5))).reshape(NE // 4, F)
    ef14 = jnp.pad(efr[:, 1::2, :], ((0, 0), (0, 0), (0, 5))).reshape(NE // 4, F)
    wz = jnp.zeros((5, F), jnp.float32)
    wd1, ws1 = W1a[:F], W1a[F:2 * F]
    we41 = jnp.kron(eye4, jnp.concatenate([W1a[2 * F:], wz], axis=0))
    w241 = jnp.kron(eye4, W2a)
    b241 = jnp.tile(b2a, 4).reshape(1, 4 * F)
    wd42 = jnp.kron(eye4, W1b[:F])
    ws42 = jnp.kron(eye4, W1b[F:2 * F])
    b142 = jnp.tile(b1b, 4).reshape(1, 4 * F)
    we42 = jnp.kron(eye4, jnp.concatenate([W1b[2 * F:], wz], axis=0))
    w242 = jnp.kron(eye4, W2b)
    b242 = jnp.tile(b2b, 4).reshape(1, 4 * F)
    wr4 = jnp.kron(eye4, Wr)
    br4 = jnp.tile(br, 4).reshape(1, 4 * F)

    bk1, be1, cn1 = _bin_edges(dst1, 1)
    bk2, be2, cn2 = _bin_edges(dst2, 2)

    ne4 = NE // 4

    wd41 = jnp.kron(eye4, wd1)
    ws41 = jnp.kron(eye4, ws1)
    b141 = jnp.tile(b1a, 4).reshape(1, 4 * F)
    h4 = jnp.reshape(h, (NN // 4, 4 * F))
    a14, b14 = _nodeproj128(h4, wd41, ws41, b141, False)
    ga1, gb1 = _gather_pairs(a14, b14, dst1, src1)
    m14 = _edge_mlp(jnp.reshape(ga1, (ne4, 4 * F)),
                    jnp.reshape(gb1, (ne4, 4 * F)), ef04, we41, w241, b241)
    h14 = _seg_max(m14, bk1, be1, cn1.reshape(-1), 1)

    a24, b24 = _nodeproj128(h14, wd42, ws42, b142, True)
    ga2, gb2 = _gather_pairs(a24, b24, dst2, src2)
    m24 = _edge_mlp(jnp.reshape(ga2, (ne4, 4 * F)),
                    jnp.reshape(gb2, (ne4, 4 * F)), ef14, we42, w242, b242)
    h2c4 = _seg_max(m24, bk2, be2, cn2.reshape(-1), 2)

    return _regression(h2c4, wr4, br4)
